# baseline (speedup 1.0000x reference)
import jax
import jax.numpy as jnp
from jax import lax
from jax.experimental import pallas as pl
from jax.experimental.pallas import tpu as pltpu
from jax.experimental.pallas import tpu_sc as plsc

_MEM_ROWS = 65536
_ASSETS = 512
_FLAT = _MEM_ROWS * _ASSETS
_NUM_WORKERS = 32
_CHUNK = _FLAT // _NUM_WORKERS


def _shift_body(new_hbm, mem_hbm, out_hbm, sem):
    cid = lax.axis_index("c")
    sid = lax.axis_index("s")
    wid = sid * 2 + cid
    base = wid * _CHUNK

    @pl.when(wid < _NUM_WORKERS - 1)
    def _full_chunk():
        cp = pltpu.make_async_copy(
            mem_hbm.at[pl.ds(base + _ASSETS, _CHUNK)],
            out_hbm.at[pl.ds(base, _CHUNK)],
            sem,
        )
        cp.start()
        cp.wait()

    @pl.when(wid == _NUM_WORKERS - 1)
    def _last_chunk():
        cp = pltpu.make_async_copy(
            mem_hbm.at[pl.ds(base + _ASSETS, _CHUNK - _ASSETS)],
            out_hbm.at[pl.ds(base, _CHUNK - _ASSETS)],
            sem,
        )
        cp.start()
        cp2 = pltpu.make_async_copy(
            new_hbm,
            out_hbm.at[pl.ds(_FLAT - _ASSETS, _ASSETS)],
            sem,
        )
        cp2.start()
        cp.wait()
        cp2.wait()


@jax.jit
def _shift(new, memory):
    mesh = plsc.VectorSubcoreMesh(core_axis_name="c", subcore_axis_name="s")
    flat = pl.kernel(
        _shift_body,
        out_type=jax.ShapeDtypeStruct((_FLAT,), jnp.float32),
        mesh=mesh,
        scratch_types=[pltpu.SemaphoreType.DMA],
    )(new, memory.reshape(_FLAT))
    return flat.reshape(_MEM_ROWS, _ASSETS)


def kernel(new, memory):
    return _shift(new, memory)


# speedup vs baseline: 12.5408x; 12.5408x over previous
import jax
import jax.numpy as jnp
from jax import lax
from jax.experimental import pallas as pl
from jax.experimental.pallas import tpu as pltpu
from jax.experimental.pallas import tpu_sc as plsc

_MEM_ROWS = 65536
_ASSETS = 512
_FLAT = _MEM_ROWS * _ASSETS
_REGION = _FLAT - _ASSETS
_NUM_WORKERS = 32
_SLAB = _REGION // _NUM_WORKERS
_NCHUNKS = 34
_B = _SLAB // _NCHUNKS
_NBUF = 4


def _shift_body(new_hbm, mem_hbm, out_hbm,
                buf0, buf1, buf2, buf3,
                isem0, isem1, isem2, isem3,
                osem0, osem1, osem2, osem3, nsem):
    cid = lax.axis_index("c")
    sid = lax.axis_index("s")
    wid = sid * 2 + cid
    base = wid * _SLAB

    bufs = (buf0, buf1, buf2, buf3)
    isems = (isem0, isem1, isem2, isem3)
    osems = (osem0, osem1, osem2, osem3)

    def gather(c):
        b = c % _NBUF
        return pltpu.make_async_copy(
            mem_hbm.at[pl.ds(base + _ASSETS + c * _B, _B)], bufs[b], isems[b])

    def scatter(c):
        b = c % _NBUF
        return pltpu.make_async_copy(
            bufs[b], out_hbm.at[pl.ds(base + c * _B, _B)], osems[b])

    new_cp = pltpu.make_async_copy(
        new_hbm, out_hbm.at[pl.ds(_REGION, _ASSETS)], nsem)

    @pl.when(wid == 0)
    def _start_new():
        new_cp.start()

    for c in range(_NBUF):
        gather(c).start()

    for c in range(_NCHUNKS):
        gather(c).wait()
        scatter(c).start()
        k = c + 2
        if _NBUF <= k < _NCHUNKS:
            scatter(k - _NBUF).wait()
            gather(k).start()

    for c in range(_NCHUNKS - _NBUF, _NCHUNKS):
        scatter(c).wait()

    @pl.when(wid == 0)
    def _wait_new():
        new_cp.wait()


@jax.jit
def _shift(new, memory):
    mesh = plsc.VectorSubcoreMesh(core_axis_name="c", subcore_axis_name="s")
    flat = pl.kernel(
        _shift_body,
        out_type=jax.ShapeDtypeStruct((_FLAT,), jnp.float32),
        mesh=mesh,
        scratch_types=[
            pltpu.VMEM((_B,), jnp.float32),
            pltpu.VMEM((_B,), jnp.float32),
            pltpu.VMEM((_B,), jnp.float32),
            pltpu.VMEM((_B,), jnp.float32),
            pltpu.SemaphoreType.DMA,
            pltpu.SemaphoreType.DMA,
            pltpu.SemaphoreType.DMA,
            pltpu.SemaphoreType.DMA,
            pltpu.SemaphoreType.DMA,
            pltpu.SemaphoreType.DMA,
            pltpu.SemaphoreType.DMA,
            pltpu.SemaphoreType.DMA,
            pltpu.SemaphoreType.DMA,
        ],
    )(new, memory.reshape(_FLAT))
    return flat.reshape(_MEM_ROWS, _ASSETS)


def kernel(new, memory):
    return _shift(new, memory)


# speedup vs baseline: 12.5993x; 1.0047x over previous
import jax
import jax.numpy as jnp
from jax import lax
from jax.experimental import pallas as pl
from jax.experimental.pallas import tpu as pltpu
from jax.experimental.pallas import tpu_sc as plsc

_MEM_ROWS = 65536
_ASSETS = 512
_FLAT = _MEM_ROWS * _ASSETS
_REGION = _FLAT - _ASSETS
_NUM_WORKERS = 32
_SLAB = _REGION // _NUM_WORKERS
_NCHUNKS = 17
_B = _SLAB // _NCHUNKS
_NBUF = 2
_LA = 1


def _shift_body(new_hbm, mem_hbm, out_hbm, *scratch):
    cid = lax.axis_index("c")
    sid = lax.axis_index("s")
    wid = sid * 2 + cid
    base = wid * _SLAB

    bufs = scratch[:_NBUF]
    isems = scratch[_NBUF:2 * _NBUF]
    osems = scratch[2 * _NBUF:3 * _NBUF]
    nsem = scratch[3 * _NBUF]

    def gather(c):
        b = c % _NBUF
        return pltpu.make_async_copy(
            mem_hbm.at[pl.ds(base + _ASSETS + c * _B, _B)], bufs[b], isems[b])

    def scatter(c):
        b = c % _NBUF
        return pltpu.make_async_copy(
            bufs[b], out_hbm.at[pl.ds(base + c * _B, _B)], osems[b])

    new_cp = pltpu.make_async_copy(
        new_hbm, out_hbm.at[pl.ds(_REGION, _ASSETS)], nsem)

    @pl.when(wid == 0)
    def _start_new():
        new_cp.start()

    for c in range(_NBUF):
        gather(c).start()

    for c in range(_NCHUNKS):
        gather(c).wait()
        scatter(c).start()
        k = c + _LA
        if _NBUF <= k < _NCHUNKS:
            scatter(k - _NBUF).wait()
            gather(k).start()

    for c in range(_NCHUNKS - _NBUF, _NCHUNKS):
        scatter(c).wait()

    @pl.when(wid == 0)
    def _wait_new():
        new_cp.wait()


@jax.jit
def _shift(new, memory):
    mesh = plsc.VectorSubcoreMesh(core_axis_name="c", subcore_axis_name="s")
    flat = pl.kernel(
        _shift_body,
        out_type=jax.ShapeDtypeStruct((_FLAT,), jnp.float32),
        mesh=mesh,
        scratch_types=(
            [pltpu.VMEM((_B,), jnp.float32)] * _NBUF
            + [pltpu.SemaphoreType.DMA] * (2 * _NBUF + 1)
        ),
    )(new, memory.reshape(_FLAT))
    return flat.reshape(_MEM_ROWS, _ASSETS)


def kernel(new, memory):
    return _shift(new, memory)
